# baseline (speedup 1.0000x reference)
import jax
import jax.numpy as jnp
from jax import lax
from jax.experimental import pallas as pl
from jax.experimental.pallas import tpu as pltpu
from jax.experimental.pallas import tpu_sc as plsc

P_L = 0.5
P_H = 0.5
LANES = 16
WORKERS = 16
BLK = 64
SUP = 8


def _prep_body(m_ref, h_ref, b_ref, out_ref):
    acc = lax.dot_general(m_ref[...], h_ref[...], (((1,), (1,)), ((), ())),
                          preferred_element_type=jnp.float32)
    out_ref[...] = acc + b_ref[:, 0:1]


def _final_body(zl_ref, zh_ref, wl_ref, wh_ref, b_ref, out_ref):
    acc = lax.dot_general(zl_ref[...], wl_ref[...], (((1,), (1,)), ((), ())),
                          preferred_element_type=jnp.float32)
    acc = acc + lax.dot_general(zh_ref[...], wh_ref[...], (((1,), (1,)), ((), ())),
                                preferred_element_type=jnp.float32)
    out_ref[...] = acc + b_ref[0:1, :]


def _make_sc_kernel(NP, D, BPW, RPW, NT, NZ, RPZ):
    mesh = plsc.VectorSubcoreMesh(core_axis_name="c", subcore_axis_name="s")
    DV = D // LANES

    def body(src_hbm, dst_hbm, h_hbm, al_hbm, bl_hbm, ah_hbm, bh_hbm,
             zl_hbm, zh_hbm, d_hbm,
             tbl_a, tbl_b, d_full, src_sb, dst_sb, rows,
             e_v, ones_v, d_sl, d_out, z_sh, deg_sh, gsem, ssem):
        c = lax.axis_index("c")
        s = lax.axis_index("s")
        zeros16 = jnp.broadcast_to((c * 0).astype(jnp.float32), (LANES,))
        ones16 = zeros16 + 1.0

        def zero_row(r, _):
            for t in range(DV):
                rows[0, r, pl.ds(t * LANES, LANES)] = zeros16
            return _
        lax.fori_loop(0, BLK, zero_row, None)
        for t in range(BLK // LANES):
            ones_v[pl.ds(t * LANES, LANES)] = ones16
            d_sl[pl.ds(t * LANES, LANES)] = zeros16
        for k in range(RPZ // BLK):
            pltpu.sync_copy(rows.at[0], z_sh.at[pl.ds(s * RPZ + k * BLK, BLK)])
        if RPZ % BLK:
            pltpu.sync_copy(rows.at[0].at[pl.ds(0, RPZ % BLK)],
                            z_sh.at[pl.ds(s * RPZ + (RPZ // BLK) * BLK, RPZ % BLK)])
        for k in range(RPW // BLK):
            pltpu.sync_copy(d_sl, deg_sh.at[pl.ds(s * RPW + k * BLK, BLK)])
        plsc.subcore_barrier()

        def deg_super(sb, _):
            pltpu.sync_copy(dst_hbm.at[pl.ds(s * BPW + sb * SUP, SUP)], dst_sb)
            hs = [pltpu.async_copy(ones_v, deg_sh.at[dst_sb.at[j]], ssem,
                                   add=True)
                  for j in range(SUP)]
            for hnd in hs:
                hnd.wait()
            return _
        lax.fori_loop(0, BPW // SUP, deg_super, None)
        plsc.subcore_barrier()

        for k in range(RPW // BLK):
            pltpu.sync_copy(deg_sh.at[pl.ds(s * RPW + k * BLK, BLK)], d_sl)
            for i in range(BLK // LANES):
                x = d_sl[pl.ds(i * LANES, LANES)] + 1.0
                bits = lax.bitcast_convert_type(x, jnp.int32)
                y = lax.bitcast_convert_type(jnp.int32(0x5F3759DF) - (bits >> 1),
                                             jnp.float32)
                for _n in range(3):
                    y = y * (1.5 - 0.5 * x * y * y)
                d_out[pl.ds(i * LANES, LANES)] = y
            pltpu.sync_copy(d_out, d_hbm.at[pl.ds(s * RPW + k * BLK, BLK)])
        is_low = c == 0

        @pl.when(is_low)
        def _():
            pltpu.sync_copy(al_hbm.at[pl.ds(0, NT)], tbl_a)
            pltpu.sync_copy(bl_hbm.at[pl.ds(0, NT)], tbl_b)

        @pl.when(jnp.logical_not(is_low))
        def _():
            pltpu.sync_copy(ah_hbm.at[pl.ds(0, NT)], tbl_a)
            pltpu.sync_copy(bh_hbm.at[pl.ds(0, NT)], tbl_b)
        plsc.subcore_barrier()
        pltpu.sync_copy(d_hbm.at[pl.ds(0, NT)], d_full)
        sign = jnp.where(is_low, 1.0, -1.0).astype(jnp.float32)
        slope = jnp.where(is_low, P_L, P_H).astype(jnp.float32)

        def gates(j):
            for g in range(BLK // LANES):
                s16 = src_sb[j, pl.ds(g * LANES, LANES)]
                t16 = dst_sb[j, pl.ds(g * LANES, LANES)]
                av = plsc.load_gather(tbl_a, [t16])
                bv = plsc.load_gather(tbl_b, [s16])
                dd = plsc.load_gather(d_full, [t16])
                dsrc = plsc.load_gather(d_full, [s16])
                x = av + bv
                t = jnp.maximum(x, -slope * x)
                u = jnp.exp(-2.0 * t)
                g16 = sign * (1.0 - u) / (1.0 + u)
                e_v[pl.ds(g * LANES, LANES)] = g16 * dd * dsrc

        def scale(p):
            def scale_group(g, _2):
                ev = e_v[pl.ds(g * LANES, LANES)]
                for jl in range(LANES):
                    es = ev[jl]
                    r = g * LANES + jl
                    for t in range(DV):
                        v = rows[p, r, pl.ds(t * LANES, LANES)]
                        rows[p, r, pl.ds(t * LANES, LANES)] = v * es
                return _2
            lax.fori_loop(0, BLK // LANES, scale_group, None)

        def edge_super(sb, _):
            pltpu.sync_copy(src_hbm.at[pl.ds(s * BPW + sb * SUP, SUP)], src_sb)
            pltpu.sync_copy(dst_hbm.at[pl.ds(s * BPW + sb * SUP, SUP)], dst_sb)
            hg = {0: pltpu.async_copy(h_hbm.at[src_sb.at[0]], rows.at[0], gsem)}
            hsc = {}
            for j in range(SUP):
                p = j % 2
                if j >= 1:
                    hsc[j - 1].wait()
                if j + 1 < SUP:
                    hg[j + 1] = pltpu.async_copy(h_hbm.at[src_sb.at[j + 1]],
                                                 rows.at[1 - p], gsem)
                gates(j)
                hg[j].wait()
                scale(p)
                hsc[j] = pltpu.async_copy(rows.at[p], z_sh.at[dst_sb.at[j]],
                                          ssem, add=True)
            hsc[SUP - 1].wait()
            return _
        lax.fori_loop(0, BPW // SUP, edge_super, None)
        plsc.subcore_barrier()

        @pl.when(is_low)
        def _():
            pltpu.sync_copy(z_sh.at[pl.ds(s * RPZ, RPZ)],
                            zl_hbm.at[pl.ds(s * RPZ, RPZ)])

        @pl.when(jnp.logical_not(is_low))
        def _():
            pltpu.sync_copy(z_sh.at[pl.ds(s * RPZ, RPZ)],
                            zh_hbm.at[pl.ds(s * RPZ, RPZ)])

    return pl.kernel(
        body,
        out_type=[jax.ShapeDtypeStruct((NZ, D), jnp.float32),
                  jax.ShapeDtypeStruct((NZ, D), jnp.float32),
                  jax.ShapeDtypeStruct((NP,), jnp.float32)],
        mesh=mesh,
        compiler_params=pltpu.CompilerParams(needs_layout_passes=False),
        scratch_types=[
            pltpu.VMEM((NT,), jnp.float32),
            pltpu.VMEM((NT,), jnp.float32),
            pltpu.VMEM((NT,), jnp.float32),
            pltpu.VMEM((SUP, BLK), jnp.int32),
            pltpu.VMEM((SUP, BLK), jnp.int32),
            pltpu.VMEM((2, BLK, D), jnp.float32),
            pltpu.VMEM((BLK,), jnp.float32),
            pltpu.VMEM((BLK,), jnp.float32),
            pltpu.VMEM((BLK,), jnp.float32),
            pltpu.VMEM((BLK,), jnp.float32),
            pltpu.VMEM_SHARED((NZ, D), jnp.float32),
            pltpu.VMEM_SHARED((NP,), jnp.float32),
            pltpu.SemaphoreType.DMA,
            pltpu.SemaphoreType.DMA,
        ],
    )


def kernel(h, edge_index, W_gl, b_gl, W_gh, b_gh, W_wrl, b_wrl):
    N, D = h.shape
    E = edge_index.shape[1]
    RPW = pl.cdiv(N, WORKERS * BLK) * BLK
    NP = WORKERS * RPW
    BPW = pl.cdiv(pl.cdiv(E, WORKERS * BLK), 8) * 8
    EP = WORKERS * BPW * BLK

    h = h.astype(jnp.float32)
    src = edge_index[0].astype(jnp.int32)
    dst = edge_index[1].astype(jnp.int32)
    src_p = jnp.concatenate([src, jnp.zeros((EP - E,), jnp.int32)])
    dst_p = jnp.concatenate([dst, jnp.full((EP - E,), N, jnp.int32)])
    src2d = src_p.reshape(WORKERS * BPW, BLK)
    dst2d = dst_p.reshape(WORKERS * BPW, BLK)

    M8 = jnp.zeros((8, D), jnp.float32)
    M8 = M8.at[0].set(W_gl[0, :D]).at[1].set(W_gl[0, D:])
    M8 = M8.at[2].set(W_gh[0, :D]).at[3].set(W_gh[0, D:])
    bias8 = jnp.zeros((8,), jnp.float32).at[0].set(b_gl[0]).at[2].set(b_gh[0])
    bias8_2d = jnp.broadcast_to(bias8[:, None], (8, D))

    h_pad = jnp.concatenate([h, jnp.zeros((NP - N, D), jnp.float32)], axis=0)
    BN = 2048
    p8 = pl.pallas_call(
        _prep_body,
        grid=(NP // BN,),
        in_specs=[pl.BlockSpec((8, D), lambda i: (0, 0)),
                  pl.BlockSpec((BN, D), lambda i: (i, 0)),
                  pl.BlockSpec((8, D), lambda i: (0, 0))],
        out_specs=pl.BlockSpec((8, BN), lambda i: (0, i)),
        out_shape=jax.ShapeDtypeStruct((8, NP), jnp.float32),
    )(M8, h_pad, bias8_2d)

    NT = pl.cdiv(N + 1, 8) * 8
    NZ = pl.cdiv(N + 1, WORKERS * 8) * WORKERS * 8
    RPZ = NZ // WORKERS
    zl, zh, _ = _make_sc_kernel(NP, D, BPW, RPW, NT, NZ, RPZ)(
        src2d, dst2d, h, p8[0], p8[1], p8[2], p8[3])

    Wl = W_wrl[:, :D]
    Wh = W_wrl[:, D:]
    bias_out = jnp.broadcast_to(b_wrl[None, :], (8, D)).astype(jnp.float32)
    BNF = NZ // 8
    out = pl.pallas_call(
        _final_body,
        grid=(NZ // BNF,),
        in_specs=[pl.BlockSpec((BNF, D), lambda i: (i, 0)),
                  pl.BlockSpec((BNF, D), lambda i: (i, 0)),
                  pl.BlockSpec((D, D), lambda i: (0, 0)),
                  pl.BlockSpec((D, D), lambda i: (0, 0)),
                  pl.BlockSpec((8, D), lambda i: (0, 0))],
        out_specs=pl.BlockSpec((BNF, D), lambda i: (i, 0)),
        out_shape=jax.ShapeDtypeStruct((NZ, D), jnp.float32),
    )(zl, zh, Wl, Wh, bias_out)
    return out[:N]


# speedup vs baseline: 1.3451x; 1.3451x over previous
import jax
import jax.numpy as jnp
from jax import lax
from jax.experimental import pallas as pl
from jax.experimental.pallas import tpu as pltpu
from jax.experimental.pallas import tpu_sc as plsc

P_L = 0.5
P_H = 0.5
LANES = 16
WORKERS = 16
BLK = 32
SUP = 8


def _prep_body(m_ref, h_ref, b_ref, out_ref):
    acc = lax.dot_general(m_ref[...], h_ref[...], (((1,), (1,)), ((), ())),
                          preferred_element_type=jnp.float32)
    out_ref[...] = acc + b_ref[:, 0:1]


def _final_body(zl_ref, zh_ref, wl_ref, wh_ref, b_ref, out_ref):
    acc = lax.dot_general(zl_ref[...], wl_ref[...], (((1,), (1,)), ((), ())),
                          preferred_element_type=jnp.float32)
    acc = acc + lax.dot_general(zh_ref[...], wh_ref[...], (((1,), (1,)), ((), ())),
                                preferred_element_type=jnp.float32)
    out_ref[...] = acc + b_ref[0:1, :]


def _make_sc_kernel(NP, D, BPW, RPW, NT, NZ, RPZ):
    mesh = plsc.VectorSubcoreMesh(core_axis_name="c", subcore_axis_name="s")
    DV = D // LANES

    def body(src_hbm, dst_hbm, h_hbm, al_hbm, bl_hbm, ah_hbm, bh_hbm,
             zl_hbm, zh_hbm, d_hbm,
             tbl_a, tbl_b, d_full, src_sb, dst_sb, rows,
             e_v, ones_v, d_sl, d_out, z_sh, deg_sh, gsem, ssem, stsem):
        c = lax.axis_index("c")
        s = lax.axis_index("s")
        zeros16 = jnp.broadcast_to((c * 0).astype(jnp.float32), (LANES,))
        ones16 = zeros16 + 1.0

        def zero_row(r, _):
            for t in range(DV):
                rows[0, r, pl.ds(t * LANES, LANES)] = zeros16
            return _
        lax.fori_loop(0, BLK, zero_row, None)
        for t in range(BLK // LANES):
            ones_v[pl.ds(t * LANES, LANES)] = ones16
            d_sl[pl.ds(t * LANES, LANES)] = zeros16
        for k in range(RPZ // BLK):
            pltpu.sync_copy(rows.at[0], z_sh.at[pl.ds(s * RPZ + k * BLK, BLK)])
        if RPZ % BLK:
            pltpu.sync_copy(rows.at[0].at[pl.ds(0, RPZ % BLK)],
                            z_sh.at[pl.ds(s * RPZ + (RPZ // BLK) * BLK, RPZ % BLK)])
        for k in range(RPW // BLK):
            pltpu.sync_copy(d_sl, deg_sh.at[pl.ds(s * RPW + k * BLK, BLK)])
        plsc.subcore_barrier()

        def deg_super(sb, _):
            pltpu.sync_copy(dst_hbm.at[pl.ds(s * BPW + sb * SUP, SUP)],
                            dst_sb.at[0])
            hs = [pltpu.async_copy(ones_v, deg_sh.at[dst_sb.at[0].at[j]], ssem,
                                   add=True)
                  for j in range(SUP)]
            for hnd in hs:
                hnd.wait()
            return _
        lax.fori_loop(0, BPW // SUP, deg_super, None)
        plsc.subcore_barrier()

        for k in range(RPW // BLK):
            pltpu.sync_copy(deg_sh.at[pl.ds(s * RPW + k * BLK, BLK)], d_sl)
            for i in range(BLK // LANES):
                x = d_sl[pl.ds(i * LANES, LANES)] + 1.0
                bits = lax.bitcast_convert_type(x, jnp.int32)
                y = lax.bitcast_convert_type(jnp.int32(0x5F3759DF) - (bits >> 1),
                                             jnp.float32)
                for _n in range(3):
                    y = y * (1.5 - 0.5 * x * y * y)
                d_out[pl.ds(i * LANES, LANES)] = y
            pltpu.sync_copy(d_out, d_hbm.at[pl.ds(s * RPW + k * BLK, BLK)])
        is_low = c == 0

        @pl.when(is_low)
        def _():
            pltpu.sync_copy(al_hbm.at[pl.ds(0, NT)], tbl_a)
            pltpu.sync_copy(bl_hbm.at[pl.ds(0, NT)], tbl_b)

        @pl.when(jnp.logical_not(is_low))
        def _():
            pltpu.sync_copy(ah_hbm.at[pl.ds(0, NT)], tbl_a)
            pltpu.sync_copy(bh_hbm.at[pl.ds(0, NT)], tbl_b)
        plsc.subcore_barrier()
        pltpu.sync_copy(d_hbm.at[pl.ds(0, NT)], d_full)
        sign = jnp.where(is_low, 1.0, -1.0).astype(jnp.float32)
        slope = jnp.where(is_low, P_L, P_H).astype(jnp.float32)

        NSB = BPW // SUP

        def gates(q, j):
            for g in range(BLK // LANES):
                s16 = src_sb[q, j, pl.ds(g * LANES, LANES)]
                t16 = dst_sb[q, j, pl.ds(g * LANES, LANES)]
                av = plsc.load_gather(tbl_a, [t16])
                bv = plsc.load_gather(tbl_b, [s16])
                dd = plsc.load_gather(d_full, [t16])
                dsrc = plsc.load_gather(d_full, [s16])
                x = av + bv
                t = jnp.maximum(x, -slope * x)
                u = jnp.exp(-2.0 * t)
                g16 = sign * (1.0 - u) / (1.0 + u)
                e_v[pl.ds(g * LANES, LANES)] = g16 * dd * dsrc

        def scale(p):
            def scale_group(g, _2):
                ev = e_v[pl.ds(g * LANES, LANES)]
                for jl in range(LANES):
                    es = ev[jl]
                    r = g * LANES + jl
                    for t in range(DV):
                        v = rows[p, r, pl.ds(t * LANES, LANES)]
                        rows[p, r, pl.ds(t * LANES, LANES)] = v * es
                return _2
            lax.fori_loop(0, BLK // LANES, scale_group, None)

        pltpu.sync_copy(src_hbm.at[pl.ds(s * BPW, SUP)], src_sb.at[0])
        pltpu.sync_copy(dst_hbm.at[pl.ds(s * BPW, SUP)], dst_sb.at[0])
        pltpu.async_copy(src_hbm.at[pl.ds(s * BPW + SUP, SUP)], src_sb.at[1],
                         stsem)
        pltpu.async_copy(dst_hbm.at[pl.ds(s * BPW + SUP, SUP)], dst_sb.at[1],
                         stsem)

        def edge_super(sb, _):
            q = sb % 2

            @pl.when(sb > 0)
            def _():
                pltpu.make_async_copy(src_hbm.at[pl.ds(s * BPW, SUP)],
                                      src_sb.at[q], stsem).wait()
                pltpu.make_async_copy(dst_hbm.at[pl.ds(s * BPW, SUP)],
                                      dst_sb.at[q], stsem).wait()

            @pl.when(sb + 1 < NSB)
            def _():
                off = s * BPW + (sb + 1) * SUP
                pltpu.async_copy(src_hbm.at[pl.ds(off, SUP)],
                                 src_sb.at[1 - q], stsem)
                pltpu.async_copy(dst_hbm.at[pl.ds(off, SUP)],
                                 dst_sb.at[1 - q], stsem)

            hg = {}
            hsc = {}
            for j in range(2):
                hg[j] = pltpu.async_copy(h_hbm.at[src_sb.at[q].at[j]],
                                         rows.at[j % 3], gsem)
            for j in range(SUP):
                p = j % 3
                if j + 2 < SUP:
                    if j >= 1:
                        hsc[j - 1].wait()
                    hg[j + 2] = pltpu.async_copy(
                        h_hbm.at[src_sb.at[q].at[j + 2]], rows.at[(j + 2) % 3],
                        gsem)
                gates(q, j)
                hg[j].wait()
                scale(p)
                hsc[j] = pltpu.async_copy(rows.at[p],
                                          z_sh.at[dst_sb.at[q].at[j]],
                                          ssem, add=True)
            for j in range(max(0, SUP - 3), SUP):
                hsc[j].wait()
            return _
        lax.fori_loop(0, NSB, edge_super, None)
        plsc.subcore_barrier()

        @pl.when(is_low)
        def _():
            pltpu.sync_copy(z_sh.at[pl.ds(s * RPZ, RPZ)],
                            zl_hbm.at[pl.ds(s * RPZ, RPZ)])

        @pl.when(jnp.logical_not(is_low))
        def _():
            pltpu.sync_copy(z_sh.at[pl.ds(s * RPZ, RPZ)],
                            zh_hbm.at[pl.ds(s * RPZ, RPZ)])

    return pl.kernel(
        body,
        out_type=[jax.ShapeDtypeStruct((NZ, D), jnp.float32),
                  jax.ShapeDtypeStruct((NZ, D), jnp.float32),
                  jax.ShapeDtypeStruct((NP,), jnp.float32)],
        mesh=mesh,
        compiler_params=pltpu.CompilerParams(needs_layout_passes=False),
        scratch_types=[
            pltpu.VMEM((NT,), jnp.float32),
            pltpu.VMEM((NT,), jnp.float32),
            pltpu.VMEM((NT,), jnp.float32),
            pltpu.VMEM((2, SUP, BLK), jnp.int32),
            pltpu.VMEM((2, SUP, BLK), jnp.int32),
            pltpu.VMEM((3, BLK, D), jnp.float32),
            pltpu.VMEM((BLK,), jnp.float32),
            pltpu.VMEM((BLK,), jnp.float32),
            pltpu.VMEM((BLK,), jnp.float32),
            pltpu.VMEM((BLK,), jnp.float32),
            pltpu.VMEM_SHARED((NZ, D), jnp.float32),
            pltpu.VMEM_SHARED((NP,), jnp.float32),
            pltpu.SemaphoreType.DMA,
            pltpu.SemaphoreType.DMA,
            pltpu.SemaphoreType.DMA,
        ],
    )


def kernel(h, edge_index, W_gl, b_gl, W_gh, b_gh, W_wrl, b_wrl):
    N, D = h.shape
    E = edge_index.shape[1]
    RPW = pl.cdiv(N, WORKERS * BLK) * BLK
    NP = WORKERS * RPW
    BPW = pl.cdiv(pl.cdiv(E, WORKERS * BLK), 8) * 8
    EP = WORKERS * BPW * BLK

    h = h.astype(jnp.float32)
    src = edge_index[0].astype(jnp.int32)
    dst = edge_index[1].astype(jnp.int32)
    src_p = jnp.concatenate([src, jnp.zeros((EP - E,), jnp.int32)])
    dst_p = jnp.concatenate([dst, jnp.full((EP - E,), N, jnp.int32)])
    src2d = src_p.reshape(WORKERS * BPW, BLK)
    dst2d = dst_p.reshape(WORKERS * BPW, BLK)

    M8 = jnp.zeros((8, D), jnp.float32)
    M8 = M8.at[0].set(W_gl[0, :D]).at[1].set(W_gl[0, D:])
    M8 = M8.at[2].set(W_gh[0, :D]).at[3].set(W_gh[0, D:])
    bias8 = jnp.zeros((8,), jnp.float32).at[0].set(b_gl[0]).at[2].set(b_gh[0])
    bias8_2d = jnp.broadcast_to(bias8[:, None], (8, D))

    h_pad = jnp.concatenate([h, jnp.zeros((NP - N, D), jnp.float32)], axis=0)
    BN = 2048
    p8 = pl.pallas_call(
        _prep_body,
        grid=(NP // BN,),
        in_specs=[pl.BlockSpec((8, D), lambda i: (0, 0)),
                  pl.BlockSpec((BN, D), lambda i: (i, 0)),
                  pl.BlockSpec((8, D), lambda i: (0, 0))],
        out_specs=pl.BlockSpec((8, BN), lambda i: (0, i)),
        out_shape=jax.ShapeDtypeStruct((8, NP), jnp.float32),
    )(M8, h_pad, bias8_2d)

    NT = pl.cdiv(N + 1, 8) * 8
    NZ = pl.cdiv(N + 1, WORKERS * 8) * WORKERS * 8
    RPZ = NZ // WORKERS
    zl, zh, _ = _make_sc_kernel(NP, D, BPW, RPW, NT, NZ, RPZ)(
        src2d, dst2d, h, p8[0], p8[1], p8[2], p8[3])

    Wl = W_wrl[:, :D]
    Wh = W_wrl[:, D:]
    bias_out = jnp.broadcast_to(b_wrl[None, :], (8, D)).astype(jnp.float32)
    BNF = NZ // 8
    out = pl.pallas_call(
        _final_body,
        grid=(NZ // BNF,),
        in_specs=[pl.BlockSpec((BNF, D), lambda i: (i, 0)),
                  pl.BlockSpec((BNF, D), lambda i: (i, 0)),
                  pl.BlockSpec((D, D), lambda i: (0, 0)),
                  pl.BlockSpec((D, D), lambda i: (0, 0)),
                  pl.BlockSpec((8, D), lambda i: (0, 0))],
        out_specs=pl.BlockSpec((BNF, D), lambda i: (i, 0)),
        out_shape=jax.ShapeDtypeStruct((NZ, D), jnp.float32),
    )(zl, zh, Wl, Wh, bias_out)
    return out[:N]
